# baseline (speedup 1.0000x reference)
import functools

import jax
import jax.numpy as jnp
from jax import lax
from jax.experimental import pallas as pl
from jax.experimental.pallas import tpu as pltpu
from jax.experimental.pallas import tpu_sc as plsc

_K = 32
_NF = 100
_TW = _K + 1


def _sc_gather_one(table, idx):
    info = plsc.get_sparse_core_info()
    nc, ns = info.num_cores, info.num_subcores
    nw = nc * ns
    b = idx.shape[0]
    bpw = b // nw
    mesh = plsc.VectorSubcoreMesh(core_axis_name="c", subcore_axis_name="s")

    @functools.partial(
        pl.kernel,
        mesh=mesh,
        out_type=jax.ShapeDtypeStruct((b, _TW), jnp.float32),
        scratch_types=[
            pltpu.VMEM((bpw,), jnp.int32),
            pltpu.VMEM((bpw, _TW), jnp.float32),
            pltpu.SemaphoreType.DMA,
        ],
    )
    def gather_kernel(t_hbm, i_hbm, o_hbm, idx_v, rows_v, sem):
        wid = lax.axis_index("s") * nc + lax.axis_index("c")
        base = wid * bpw
        pltpu.sync_copy(i_hbm.at[pl.ds(base, bpw)], idx_v)

        def issue_group(g, carry):
            vec = idx_v[pl.ds(g * 16, 16)]
            for k in range(16):
                j = g * 16 + k
                pltpu.async_copy(t_hbm.at[pl.ds(vec[k], 1)],
                                 rows_v.at[pl.ds(j, 1)], sem)
            return carry

        lax.fori_loop(0, bpw // 16, issue_group, 0)
        pltpu.make_async_copy(t_hbm.at[pl.ds(0, bpw)], rows_v, sem).wait()
        pltpu.sync_copy(rows_v, o_hbm.at[pl.ds(base, bpw)])

    return gather_kernel(table, idx)


def _build_weights(feat_table):
    vf = feat_table[:, :_K]
    wf = feat_table[:, _K]
    d = (2 + _NF) * _K
    kx = 2 * _TW + _NF
    wv = jnp.zeros((kx, d), jnp.float32)
    wv = wv.at[jnp.arange(_K), jnp.arange(_K)].set(1.0)
    wv = wv.at[_TW + jnp.arange(_K), _K + jnp.arange(_K)].set(1.0)
    rows = 2 * _TW + jnp.repeat(jnp.arange(_NF), _K)
    cols = 2 * _K + jnp.arange(_NF * _K)
    wv = wv.at[rows, cols].set(vf.reshape(-1))
    ww = jnp.zeros((kx, 2 + _NF), jnp.float32)
    ww = ww.at[_K, 0].set(1.0)
    ww = ww.at[_TW + _K, 1].set(1.0)
    ww = ww.at[2 * _TW + jnp.arange(_NF), 2 + jnp.arange(_NF)].set(wf)
    eye = jnp.eye(_K, dtype=jnp.float32)
    zrow = jnp.zeros((1, _K), jnp.float32)
    ws = jnp.concatenate([eye, zrow, eye, zrow, vf], axis=0)
    wq = jnp.concatenate([eye, zrow, eye, zrow, vf * vf], axis=0)
    return wv, ww, ws, wq


def _fm_body(x_ref, wv_hi_ref, wv_lo_ref, ww_ref, ws_ref, wq_ref,
             v_ref, w_ref, s_ref):
    x = x_ref[...]
    x_hi = x.astype(jnp.bfloat16)
    x_lo = (x - x_hi.astype(jnp.float32)).astype(jnp.bfloat16)
    wv_hi = wv_hi_ref[...]
    wv_lo = wv_lo_ref[...]
    acc = jnp.dot(x_hi, wv_hi, preferred_element_type=jnp.float32)
    acc = acc + jnp.dot(x_hi, wv_lo, preferred_element_type=jnp.float32)
    acc = acc + jnp.dot(x_lo, wv_hi, preferred_element_type=jnp.float32)
    v_ref[...] = acc
    p = lax.Precision.HIGHEST
    w_blk = jnp.dot(x, ww_ref[...], precision=p,
                    preferred_element_type=jnp.float32)
    w_ref[...] = w_blk
    s_sum = jnp.dot(x, ws_ref[...], precision=p,
                    preferred_element_type=jnp.float32)
    s_sq = jnp.dot(x * x, wq_ref[...], precision=p,
                   preferred_element_type=jnp.float32)
    s_ref[...] = (jnp.sum(w_blk, axis=1)
                  + 0.5 * jnp.sum(s_sum * s_sum - s_sq, axis=1))[:, None]


def kernel(u, i, feats, user_table, item_table, feat_table, w0):
    b = feats.shape[0]
    u_idx = u.reshape(b).astype(jnp.int32)
    i_idx = i.reshape(b).astype(jnp.int32)
    u_rows = _sc_gather_one(user_table, u_idx)
    i_rows = _sc_gather_one(item_table, i_idx)
    x = jnp.concatenate([u_rows, i_rows, feats], axis=1)
    wv, ww, ws, wq = _build_weights(feat_table)
    wv_hi = wv.astype(jnp.bfloat16)
    wv_lo = (wv - wv_hi.astype(jnp.float32)).astype(jnp.bfloat16)
    kx, d = wv.shape
    bb = 256
    vflat, w, s2 = pl.pallas_call(
        _fm_body,
        grid=(b // bb,),
        in_specs=[
            pl.BlockSpec((bb, kx), lambda g: (g, 0)),
            pl.BlockSpec((kx, d), lambda g: (0, 0)),
            pl.BlockSpec((kx, d), lambda g: (0, 0)),
            pl.BlockSpec((kx, 2 + _NF), lambda g: (0, 0)),
            pl.BlockSpec((kx, _K), lambda g: (0, 0)),
            pl.BlockSpec((kx, _K), lambda g: (0, 0)),
        ],
        out_specs=[
            pl.BlockSpec((bb, d), lambda g: (g, 0)),
            pl.BlockSpec((bb, 2 + _NF), lambda g: (g, 0)),
            pl.BlockSpec((bb, 1), lambda g: (g, 0)),
        ],
        out_shape=[
            jax.ShapeDtypeStruct((b, d), jnp.float32),
            jax.ShapeDtypeStruct((b, 2 + _NF), jnp.float32),
            jax.ShapeDtypeStruct((b, 1), jnp.float32),
        ],
        compiler_params=pltpu.CompilerParams(
            dimension_semantics=("parallel",)),
    )(x, wv_hi, wv_lo, ww, ws, wq)
    s = s2.reshape(b) + w0
    v = vflat.reshape(b, 2 + _NF, _K)
    return (s, w, v)
